# baseline (speedup 1.0000x reference)
import jax
import jax.numpy as jnp
from jax.experimental import pallas as pl

BB = 1


def _tc_body(am_ref, pt_ref, out_ref):
    L = pt_ref.shape[0]
    for bb in range(BB):
        a = am_ref[:, bb]
        d = a[:, :, 1] - a[:, :, 0]
        d = jnp.sum(d, axis=0, keepdims=True) * (1.0 / L)
        w = jax.nn.sigmoid(d)

        acc = jnp.zeros((1, pt_ref.shape[3]), dtype=jnp.float32)
        for l in range(L):
            acc = acc + jnp.dot(w, pt_ref[l, bb],
                                preferred_element_type=jnp.float32)
        s = acc * (1.0 / L)

        n = jnp.sqrt(jnp.sum(s * s, axis=1, keepdims=True))
        out_ref[0, bb] = (s / jnp.maximum(n, 1e-12))[0]


def kernel(patch_tokens, anomaly_maps):
    L, B, T, D = patch_tokens.shape

    out = pl.pallas_call(
        _tc_body,
        grid=(B // BB,),
        in_specs=[
            pl.BlockSpec((L, BB, T, 2), lambda b: (0, b, 0, 0)),
            pl.BlockSpec((L, BB, T, D), lambda b: (0, b, 0, 0)),
        ],
        out_specs=pl.BlockSpec((1, BB, D), lambda b: (b, 0, 0)),
        out_shape=jax.ShapeDtypeStruct((B // BB, BB, D), jnp.float32),
    )(anomaly_maps, patch_tokens)
    return out.reshape(B, D)


# speedup vs baseline: 1.4476x; 1.4476x over previous
import jax
import jax.numpy as jnp
from jax.experimental import pallas as pl


def _tc_body(am_ref, pt0_ref, pt1_ref, out_ref):
    a = am_ref[:, 0]
    d = a[:, 1, :] - a[:, 0, :]
    d = jnp.sum(d, axis=0, keepdims=True) * 0.25
    w = jax.nn.sigmoid(d)

    L = 2 * pt0_ref.shape[0]
    acc = jnp.zeros((1, pt0_ref.shape[3]), dtype=jnp.float32)
    for ref in (pt0_ref, pt1_ref):
        for l in range(ref.shape[0]):
            acc = acc + jnp.dot(w, ref[l, 0], preferred_element_type=jnp.float32)
    s = acc * (1.0 / L)

    n = jnp.sqrt(jnp.sum(s * s, axis=1, keepdims=True))
    out_ref[...] = (s / jnp.maximum(n, 1e-12))[:, None, :]


def kernel(patch_tokens, anomaly_maps):
    L, B, T, D = patch_tokens.shape
    am_t = jnp.swapaxes(anomaly_maps, 2, 3)

    out = pl.pallas_call(
        _tc_body,
        grid=(B,),
        in_specs=[
            pl.BlockSpec((L, 1, 2, T), lambda b: (0, b, 0, 0)),
            pl.BlockSpec((L // 2, 1, T, D), lambda b: (0, b, 0, 0)),
            pl.BlockSpec((L // 2, 1, T, D), lambda b: (1, b, 0, 0)),
        ],
        out_specs=pl.BlockSpec((1, 1, D), lambda b: (b, 0, 0)),
        out_shape=jax.ShapeDtypeStruct((B, 1, D), jnp.float32),
    )(am_t, patch_tokens, patch_tokens)
    return out.reshape(B, D)


# speedup vs baseline: 1.4527x; 1.0036x over previous
import jax
import jax.numpy as jnp
from jax.experimental import pallas as pl


def _tc_body(am_ref, pt_ref, out_ref):
    a = am_ref[:, 0]
    d = a[:, 1, :] - a[:, 0, :]
    d = jnp.sum(d, axis=0, keepdims=True) * 0.25
    w = jax.nn.sigmoid(d)

    L = pt_ref.shape[0]
    acc = jnp.zeros((1, pt_ref.shape[3]), dtype=jnp.float32)
    for l in range(L):
        acc = acc + jnp.dot(w, pt_ref[l, 0], preferred_element_type=jnp.float32)
    s = acc * (1.0 / L)

    n = jnp.sqrt(jnp.sum(s * s, axis=1, keepdims=True))
    out_ref[...] = (s / jnp.maximum(n, 1e-12))[:, None, :]


def kernel(patch_tokens, anomaly_maps):
    L, B, T, D = patch_tokens.shape
    am_t = jnp.swapaxes(anomaly_maps, 2, 3)

    out = pl.pallas_call(
        _tc_body,
        grid=(B,),
        in_specs=[
            pl.BlockSpec((L, 1, 2, T), lambda b: (0, b, 0, 0)),
            pl.BlockSpec((L, 1, T, D), lambda b: (0, b, 0, 0)),
        ],
        out_specs=pl.BlockSpec((1, 1, D), lambda b: (b, 0, 0)),
        out_shape=jax.ShapeDtypeStruct((B, 1, D), jnp.float32),
    )(am_t, patch_tokens)
    return out.reshape(B, D)
